# Initial kernel scaffold; baseline (speedup 1.0000x reference)
import jax
import jax.numpy as jnp
from jax.experimental import pallas as pl


def kernel(x, edge_index, edge_attr, params):
    raise NotImplementedError("write your pallas kernel here")


# speedup vs baseline: 2.0168x; 2.0168x over previous
import functools
import math

import jax
import jax.numpy as jnp
import numpy as np
from jax.experimental import pallas as pl
from jax.experimental.pallas import tpu as pltpu

_F32 = jnp.float32


def _gelu(x):
    return x * 0.5 * (1.0 + jax.lax.erf(x * (2.0 ** -0.5)))


def _ln(x, g, b, eps=1e-5):
    m = jnp.mean(x, axis=-1, keepdims=True)
    v = jnp.mean((x - m) ** 2, axis=-1, keepdims=True)
    return (x - m) * jax.lax.rsqrt(v + eps) * g + b


def _row2(v):
    return v.reshape(1, -1).astype(_F32)


def _node_pre_body(x_ref, bmat_ref, w1x_ref, w1f_ref, b1_ref, lng_ref,
                   lnb_ref, w2_ref, b2_ref, h_ref):
    x = x_ref[...]
    bm = bmat_ref[...]
    xb = x.astype(jnp.bfloat16).astype(_F32)
    bmb = bm.astype(jnp.bfloat16).astype(_F32)
    proj = (2.0 * math.pi) * (xb[:, 0:1] * bmb[0:1, :] + xb[:, 1:2] * bmb[1:2, :])
    ff = jnp.concatenate([jnp.sin(proj), jnp.cos(proj)], axis=1)
    pre = (jnp.dot(x, w1x_ref[...], preferred_element_type=_F32)
           + jnp.dot(ff, w1f_ref[...], preferred_element_type=_F32)
           + b1_ref[...])
    t = _gelu(_ln(pre, lng_ref[...], lnb_ref[...]))
    h_ref[...] = jnp.dot(t, w2_ref[...], preferred_element_type=_F32) + b2_ref[...]


def _layer_mm_body(h_ref, w_ref, deg_ref, xws_ref):
    xw = jnp.dot(h_ref[...], w_ref[...], preferred_element_type=_F32)
    dinv = jax.lax.rsqrt(jnp.maximum(deg_ref[...], 1e-12))
    xws_ref[...] = xw * dinv


def _layer_post_body(agg_ref, xws_ref, deg_ref, hin_ref, hfin_ref, b_ref,
                     g_ref, bb_ref, alpha_ref, hout_ref, hfinout_ref):
    dinv = jax.lax.rsqrt(jnp.maximum(deg_ref[...], 1e-12))
    pre = (agg_ref[...] + xws_ref[...]) * dinv + b_ref[...]
    h_out = _gelu(_ln(pre, g_ref[...], bb_ref[...])) + hin_ref[...]
    hout_ref[...] = h_out
    hfinout_ref[...] = hfin_ref[...] + alpha_ref[0, 0] * h_out


def _heads_body(hfin_ref, orw1_ref, orb1_ref, orw2_ref, orb2_ref, ws_ref,
                wd_ref, reg_ref, ps_ref, pd_ref):
    hf = hfin_ref[...]
    r = _gelu(jnp.dot(hf, orw1_ref[...], preferred_element_type=_F32) + orb1_ref[...])
    reg = jnp.dot(r, orw2_ref[...], preferred_element_type=_F32) + orb2_ref[...]
    off = jnp.clip(reg, -100.0, 100.0)
    unc = jnp.clip(jax.nn.softplus(reg), 0.01, 10.0)
    col = jax.lax.broadcasted_iota(jnp.int32, reg.shape, 1)
    reg_ref[...] = jnp.where(col < 2, off, unc)
    ps_ref[...] = jnp.dot(hf, ws_ref[...], preferred_element_type=_F32)
    pd_ref[...] = jnp.dot(hf, wd_ref[...], preferred_element_type=_F32)


def _edge_pre_body(ea_ref, w1_ref, b1_ref, lng_ref, lnb_ref, w2_ref, b2_ref,
                   we_ref, ce_ref, ec_ref):
    pre = jnp.dot(ea_ref[...], w1_ref[...], preferred_element_type=_F32) + b1_ref[...]
    t = _gelu(_ln(pre, lng_ref[...], lnb_ref[...]))
    ee = jnp.dot(t, w2_ref[...], preferred_element_type=_F32) + b2_ref[...]
    ec_ref[...] = jnp.dot(ee, we_ref[...], preferred_element_type=_F32) + ce_ref[...]


def _edge_score_body(g1_ref, g2_ref, ec_ref, w2_ref, b2_ref, out_ref):
    s = _gelu(g1_ref[...] + g2_ref[...] + ec_ref[...])
    logit = jnp.dot(s, w2_ref[...], preferred_element_type=_F32) + b2_ref[...]
    out_ref[...] = jax.nn.sigmoid(logit)


def _full(shape):
    return pl.BlockSpec(shape, lambda i: tuple(0 for _ in shape))


def _rows(bn, d):
    return pl.BlockSpec((bn, d), lambda i: (i, 0))


def kernel(x, edge_index, edge_attr, params):
    n, d_node = x.shape
    e = edge_attr.shape[0]
    hid = params['node_w2'].shape[0]
    nl = params['conv_w'].shape[0]
    src = edge_index[0]
    dst = edge_index[1]

    alpha = jax.nn.softmax(jnp.mean(params['attn_vector'], axis=1), axis=0)
    w_s = params['ec_w1'][0:hid]
    w_d = params['ec_w1'][hid:2 * hid]
    w_e = params['ec_w1'][2 * hid:3 * hid]
    c_e = params['ec_b1']
    bmat = jnp.zeros((8, 32), _F32).at[0:2].set(params['B'])
    w1x = params['node_w1'][:d_node]
    w1f = params['node_w1'][d_node:]
    orw2 = jnp.zeros((hid, 8), _F32).at[:, 0:4].set(params['or_w2'])
    orb2 = jnp.zeros((1, 8), _F32).at[0, 0:4].set(params['or_b2'])
    ecw2 = jnp.zeros((hid, 8), _F32).at[:, 0:1].set(params['ec_w2'])
    ecb2 = jnp.zeros((1, 8), _F32).at[0, 0:1].set(params['ec_b2'])

    bn = 1000
    be = 2000
    gn = n // bn
    ge = e // be

    deg = jax.ops.segment_sum(jnp.ones((e,), _F32), dst, num_segments=n) + 1.0
    deg2 = deg.reshape(n, 1)

    h = pl.pallas_call(
        _node_pre_body,
        grid=(gn,),
        in_specs=[_rows(bn, d_node), _full((8, 32)), _full((d_node, hid)),
                  _full((64, hid)), _full((1, hid)), _full((1, hid)),
                  _full((1, hid)), _full((hid, hid)), _full((1, hid))],
        out_specs=_rows(bn, hid),
        out_shape=jax.ShapeDtypeStruct((n, hid), _F32),
    )(x, bmat, w1x, w1f, _row2(params['node_b1']), _row2(params['node_ln_g']),
      _row2(params['node_ln_b']), params['node_w2'], _row2(params['node_b2']))

    e_contrib = pl.pallas_call(
        _edge_pre_body,
        grid=(ge,),
        in_specs=[_rows(be, 16), _full((16, hid)), _full((1, hid)),
                  _full((1, hid)), _full((1, hid)), _full((hid, hid)),
                  _full((1, hid)), _full((hid, hid)), _full((1, hid))],
        out_specs=_rows(be, hid),
        out_shape=jax.ShapeDtypeStruct((e, hid), _F32),
    )(edge_attr, params['edge_w1'], _row2(params['edge_b1']),
      _row2(params['edge_ln_g']), _row2(params['edge_ln_b']),
      params['edge_w2'], _row2(params['edge_b2']), w_e, _row2(c_e))

    h_in = h
    h_fin = jnp.zeros((n, hid), _F32)
    for i in range(nl):
        xws = pl.pallas_call(
            _layer_mm_body,
            grid=(gn,),
            in_specs=[_rows(bn, hid), _full((hid, hid)), _rows(bn, 1)],
            out_specs=_rows(bn, hid),
            out_shape=jax.ShapeDtypeStruct((n, hid), _F32),
        )(h_in, params['conv_w'][i], deg2)

        agg = jax.ops.segment_sum(xws[src], dst, num_segments=n)

        h_in, h_fin = pl.pallas_call(
            _layer_post_body,
            grid=(gn,),
            in_specs=[_rows(bn, hid), _rows(bn, hid), _rows(bn, 1),
                      _rows(bn, hid), _rows(bn, hid), _full((1, hid)),
                      _full((1, hid)), _full((1, hid)), _full((1, 1))],
            out_specs=[_rows(bn, hid), _rows(bn, hid)],
            out_shape=[jax.ShapeDtypeStruct((n, hid), _F32),
                       jax.ShapeDtypeStruct((n, hid), _F32)],
        )(agg, xws, deg2, h_in, h_fin, _row2(params['conv_b'][i]),
          _row2(params['bn_g'][i]), _row2(params['bn_b'][i]),
          alpha[i].reshape(1, 1))

    reg, p_s, p_d = pl.pallas_call(
        _heads_body,
        grid=(gn,),
        in_specs=[_rows(bn, hid), _full((hid, hid)), _full((1, hid)),
                  _full((hid, 8)), _full((1, 8)), _full((hid, hid)),
                  _full((hid, hid))],
        out_specs=[_rows(bn, 8), _rows(bn, hid), _rows(bn, hid)],
        out_shape=[jax.ShapeDtypeStruct((n, 8), _F32),
                   jax.ShapeDtypeStruct((n, hid), _F32),
                   jax.ShapeDtypeStruct((n, hid), _F32)],
    )(h_fin, params['or_w1'], _row2(params['or_b1']), orw2, orb2, w_s, w_d)

    g1 = p_s[src]
    g2 = p_d[dst]

    scores8 = pl.pallas_call(
        _edge_score_body,
        grid=(ge,),
        in_specs=[_rows(be, hid), _rows(be, hid), _rows(be, hid),
                  _full((hid, 8)), _full((1, 8))],
        out_specs=_rows(be, 8),
        out_shape=jax.ShapeDtypeStruct((e, 8), _F32),
    )(g1, g2, e_contrib, ecw2, ecb2)

    edge_scores = scores8[:, 0]
    pred_offsets = reg[:, 0:2]
    pred_uncertainty = reg[:, 2:4]
    return (edge_scores, pred_offsets, pred_uncertainty, h_fin)


# speedup vs baseline: 3.6189x; 1.7943x over previous
import functools
import math

import jax
import jax.numpy as jnp
import numpy as np
from jax.experimental import pallas as pl
from jax.experimental.pallas import tpu as pltpu
from jax.experimental.pallas import tpu_sc as plsc

_F32 = jnp.float32


def _gelu(x):
    return x * 0.5 * (1.0 + jax.lax.erf(x * (2.0 ** -0.5)))


def _ln(x, g, b, eps=1e-5):
    m = jnp.mean(x, axis=-1, keepdims=True)
    v = jnp.mean((x - m) ** 2, axis=-1, keepdims=True)
    return (x - m) * jax.lax.rsqrt(v + eps) * g + b


def _row2(v):
    return v.reshape(1, -1).astype(_F32)


def _node_pre_body(x_ref, bmat_ref, w1x_ref, w1f_ref, b1_ref, lng_ref,
                   lnb_ref, w2_ref, b2_ref, h_ref):
    x = x_ref[...]
    bm = bmat_ref[...]
    xb = x.astype(jnp.bfloat16).astype(_F32)
    bmb = bm.astype(jnp.bfloat16).astype(_F32)
    proj = (2.0 * math.pi) * (xb[:, 0:1] * bmb[0:1, :] + xb[:, 1:2] * bmb[1:2, :])
    ff = jnp.concatenate([jnp.sin(proj), jnp.cos(proj)], axis=1)
    pre = (jnp.dot(x, w1x_ref[...], preferred_element_type=_F32)
           + jnp.dot(ff, w1f_ref[...], preferred_element_type=_F32)
           + b1_ref[...])
    t = _gelu(_ln(pre, lng_ref[...], lnb_ref[...]))
    h_ref[...] = jnp.dot(t, w2_ref[...], preferred_element_type=_F32) + b2_ref[...]


def _dinv_from_parts(deg_ref):
    deg = deg_ref[0, :, 0:1] + deg_ref[1, :, 0:1] + 1.0
    return jax.lax.rsqrt(jnp.maximum(deg, 1e-12))


def _layer_mm_body(h_ref, w_ref, deg_ref, *x_refs):
    xw = jnp.dot(h_ref[...], w_ref[...], preferred_element_type=_F32)
    dinv = _dinv_from_parts(deg_ref)
    xws = xw * dinv
    for k, xr in enumerate(x_refs):
        xr[...] = xws[:, 16 * k:16 * k + 16]


def _layer_post_body(*refs):
    a_refs = refs[0:6]
    x_refs = refs[6:12]
    (deg_ref, hin_ref, hfin_ref, b_ref, g_ref, bb_ref, alpha_ref,
     hout_ref, hfinout_ref) = refs[12:]
    dinv = _dinv_from_parts(deg_ref)
    agg = jnp.concatenate([a[0] + a[1] for a in a_refs], axis=-1)
    xws = jnp.concatenate([x[...] for x in x_refs], axis=-1)
    pre = (agg + xws) * dinv + b_ref[...]
    h_out = _gelu(_ln(pre, g_ref[...], bb_ref[...])) + hin_ref[...]
    hout_ref[...] = h_out
    hfinout_ref[...] = hfin_ref[...] + alpha_ref[0, 0] * h_out


_SC_NC = 2
_SC_NS = 16
_SC_RPS = 200
_SC_GRP = 5
_SC_EPAD = _SC_NC * _SC_NS * _SC_RPS * 128
_SC_NACC = 50176


def _sc_agg_body(t0, t1, t2, t3, t4, t5, src2d, dst2d, zer,
                 o0, o1, o2, o3, o4, o5,
                 srcidx, dstidx, buf0, buf1, accum,
                 semg0, semg1, sems0, sems1):
    cid = jax.lax.axis_index("c")
    sid = jax.lax.axis_index("s")
    rowbase = (cid * _SC_NS + sid) * _SC_RPS
    accrows = _SC_NACC // _SC_NS
    accbase = sid * accrows

    pltpu.sync_copy(src2d.at[pl.ds(rowbase, _SC_RPS)], srcidx)
    pltpu.sync_copy(dst2d.at[pl.ds(rowbase, _SC_RPS)], dstidx)

    for table, out in ((t0, o0), (t1, o1), (t2, o2), (t3, o3), (t4, o4), (t5, o5)):
        pltpu.sync_copy(zer, accum.at[pl.ds(accbase, accrows)])
        plsc.subcore_barrier()

        def pair(i, carry):
            base0 = i * (2 * _SC_GRP)
            d0 = [pltpu.async_copy(table.at[srcidx.at[base0 + j]],
                                   buf0.at[pl.ds(j * 128, 128)], semg0)
                  for j in range(_SC_GRP)]
            for d in d0:
                d.wait()
            d1 = [pltpu.async_copy(table.at[srcidx.at[base0 + _SC_GRP + j]],
                                   buf1.at[pl.ds(j * 128, 128)], semg1)
                  for j in range(_SC_GRP)]
            s0 = [pltpu.async_copy(buf0.at[pl.ds(j * 128, 128)],
                                   accum.at[dstidx.at[base0 + j]], sems0,
                                   add=True)
                  for j in range(_SC_GRP)]
            for d in d1:
                d.wait()
            for d in s0:
                d.wait()
            s1 = [pltpu.async_copy(buf1.at[pl.ds(j * 128, 128)],
                                   accum.at[dstidx.at[base0 + _SC_GRP + j]],
                                   sems1, add=True)
                  for j in range(_SC_GRP)]
            for d in s1:
                d.wait()
            return carry

        jax.lax.fori_loop(0, _SC_RPS // (2 * _SC_GRP), pair, 0)
        plsc.subcore_barrier()
        pltpu.sync_copy(accum.at[pl.ds(accbase, accrows)],
                        out.at[cid, pl.ds(accbase, accrows)])


def _sc_deg_body(dst2d, ones16, zer16, odeg, dstidx, buf, accum, semd):
    cid = jax.lax.axis_index("c")
    sid = jax.lax.axis_index("s")
    rowbase = (cid * _SC_NS + sid) * _SC_RPS
    accrows = _SC_NACC // _SC_NS
    accbase = sid * accrows

    pltpu.sync_copy(dst2d.at[pl.ds(rowbase, _SC_RPS)], dstidx)
    pltpu.sync_copy(ones16, buf)
    pltpu.sync_copy(zer16, accum.at[pl.ds(accbase, accrows)])
    plsc.subcore_barrier()

    def grp(i, carry):
        base0 = i * _SC_GRP
        ds = [pltpu.async_copy(buf.at[pl.ds(j * 128, 128)],
                               accum.at[dstidx.at[base0 + j]], semd,
                               add=True)
              for j in range(_SC_GRP)]
        for d in ds:
            d.wait()
        return carry

    jax.lax.fori_loop(0, _SC_RPS // _SC_GRP, grp, 0)
    plsc.subcore_barrier()
    pltpu.sync_copy(accum.at[pl.ds(accbase, accrows)],
                    odeg.at[cid, pl.ds(accbase, accrows)])


def _sc_deg(dst2d, ones16, zer16):
    mesh = plsc.VectorSubcoreMesh(core_axis_name="c", subcore_axis_name="s")
    f = pl.kernel(
        _sc_deg_body,
        out_type=jax.ShapeDtypeStruct((_SC_NC, _SC_NACC, 16), _F32),
        mesh=mesh,
        scratch_types=[
            pltpu.VMEM((_SC_RPS, 128), jnp.int32),
            pltpu.VMEM((_SC_GRP * 128, 16), _F32),
            pltpu.VMEM_SHARED((_SC_NACC, 16), _F32),
            pltpu.SemaphoreType.DMA,
        ],
        compiler_params=pltpu.CompilerParams(use_tc_tiling_on_sc=False),
    )
    return f(dst2d, ones16, zer16)


def _sc_agg(xws_chunks, src2d, dst2d, zer):
    n_acc = _SC_NACC
    mesh = plsc.VectorSubcoreMesh(core_axis_name="c", subcore_axis_name="s")
    f = pl.kernel(
        _sc_agg_body,
        out_type=[jax.ShapeDtypeStruct((_SC_NC, n_acc, 16), _F32)] * 6,
        mesh=mesh,
        scratch_types=[
            pltpu.VMEM((_SC_RPS, 128), jnp.int32),
            pltpu.VMEM((_SC_RPS, 128), jnp.int32),
            pltpu.VMEM((_SC_GRP * 128, 16), _F32),
            pltpu.VMEM((_SC_GRP * 128, 16), _F32),
            pltpu.VMEM_SHARED((n_acc, 16), _F32),
            pltpu.SemaphoreType.DMA,
            pltpu.SemaphoreType.DMA,
            pltpu.SemaphoreType.DMA,
            pltpu.SemaphoreType.DMA,
        ],
        compiler_params=pltpu.CompilerParams(use_tc_tiling_on_sc=False),
    )
    return f(*xws_chunks, src2d, dst2d, zer)


def _heads_body(hfin_ref, orw1_ref, orb1_ref, orw2_ref, orb2_ref, ws_ref,
                wd_ref, reg_ref, ps_ref, pd_ref):
    hf = hfin_ref[...]
    r = _gelu(jnp.dot(hf, orw1_ref[...], preferred_element_type=_F32) + orb1_ref[...])
    reg = jnp.dot(r, orw2_ref[...], preferred_element_type=_F32) + orb2_ref[...]
    off = jnp.clip(reg, -100.0, 100.0)
    unc = jnp.clip(jax.nn.softplus(reg), 0.01, 10.0)
    col = jax.lax.broadcasted_iota(jnp.int32, reg.shape, 1)
    reg_ref[...] = jnp.where(col < 2, off, unc)
    ps_ref[...] = jnp.dot(hf, ws_ref[...], preferred_element_type=_F32)
    pd_ref[...] = jnp.dot(hf, wd_ref[...], preferred_element_type=_F32)


def _edge_pre_body(ea_ref, w1_ref, b1_ref, lng_ref, lnb_ref, w2_ref, b2_ref,
                   we_ref, ce_ref, ec_ref):
    pre = jnp.dot(ea_ref[...], w1_ref[...], preferred_element_type=_F32) + b1_ref[...]
    t = _gelu(_ln(pre, lng_ref[...], lnb_ref[...]))
    ee = jnp.dot(t, w2_ref[...], preferred_element_type=_F32) + b2_ref[...]
    ec_ref[...] = jnp.dot(ee, we_ref[...], preferred_element_type=_F32) + ce_ref[...]


def _edge_score_body(g1_ref, g2_ref, ec_ref, w2_ref, b2_ref, out_ref):
    s = _gelu(g1_ref[...] + g2_ref[...] + ec_ref[...])
    logit = jnp.dot(s, w2_ref[...], preferred_element_type=_F32) + b2_ref[...]
    out_ref[...] = jax.nn.sigmoid(logit)


def _full(shape):
    return pl.BlockSpec(shape, lambda i: tuple(0 for _ in shape))


def _rows(bn, d):
    return pl.BlockSpec((bn, d), lambda i: (i, 0))


def kernel(x, edge_index, edge_attr, params):
    n, d_node = x.shape
    e = edge_attr.shape[0]
    hid = params['node_w2'].shape[0]
    nl = params['conv_w'].shape[0]
    src = edge_index[0]
    dst = edge_index[1]

    alpha = jax.nn.softmax(jnp.mean(params['attn_vector'], axis=1), axis=0)
    w_s = params['ec_w1'][0:hid]
    w_d = params['ec_w1'][hid:2 * hid]
    w_e = params['ec_w1'][2 * hid:3 * hid]
    c_e = params['ec_b1']
    bmat = jnp.zeros((8, 32), _F32).at[0:2].set(params['B'])
    w1x = params['node_w1'][:d_node]
    w1f = params['node_w1'][d_node:]
    orw2 = jnp.zeros((hid, 8), _F32).at[:, 0:4].set(params['or_w2'])
    orb2 = jnp.zeros((1, 8), _F32).at[0, 0:4].set(params['or_b2'])
    ecw2 = jnp.zeros((hid, 8), _F32).at[:, 0:1].set(params['ec_w2'])
    ecb2 = jnp.zeros((1, 8), _F32).at[0, 0:1].set(params['ec_b2'])

    bn = 1000
    be = 2000
    gn = n // bn
    ge = e // be

    npadrow = _SC_EPAD - e
    src2d = jnp.concatenate(
        [src, jnp.zeros((npadrow,), jnp.int32)]).reshape(-1, 128)
    dst2d = jnp.concatenate(
        [dst, jnp.full((npadrow,), n, jnp.int32)]).reshape(-1, 128)
    zer = jnp.zeros((_SC_NACC // _SC_NS, 16), _F32)
    zer16 = jnp.zeros((_SC_NACC // _SC_NS, 16), _F32)
    ones16 = jnp.zeros((_SC_GRP * 128, 16), _F32).at[:, 0].set(1.0)

    deg_parts = _sc_deg(dst2d, ones16, zer16)

    def _deg_spec(bn_):
        return pl.BlockSpec((_SC_NC, bn_, 16), lambda i: (0, i, 0))

    h = pl.pallas_call(
        _node_pre_body,
        grid=(gn,),
        in_specs=[_rows(bn, d_node), _full((8, 32)), _full((d_node, hid)),
                  _full((64, hid)), _full((1, hid)), _full((1, hid)),
                  _full((1, hid)), _full((hid, hid)), _full((1, hid))],
        out_specs=_rows(bn, hid),
        out_shape=jax.ShapeDtypeStruct((n, hid), _F32),
    )(x, bmat, w1x, w1f, _row2(params['node_b1']), _row2(params['node_ln_g']),
      _row2(params['node_ln_b']), params['node_w2'], _row2(params['node_b2']))

    e_contrib = pl.pallas_call(
        _edge_pre_body,
        grid=(ge,),
        in_specs=[_rows(be, 16), _full((16, hid)), _full((1, hid)),
                  _full((1, hid)), _full((1, hid)), _full((hid, hid)),
                  _full((1, hid)), _full((hid, hid)), _full((1, hid))],
        out_specs=_rows(be, hid),
        out_shape=jax.ShapeDtypeStruct((e, hid), _F32),
    )(edge_attr, params['edge_w1'], _row2(params['edge_b1']),
      _row2(params['edge_ln_g']), _row2(params['edge_ln_b']),
      params['edge_w2'], _row2(params['edge_b2']), w_e, _row2(c_e))

    def _part_spec():
        return pl.BlockSpec((_SC_NC, bn, 16), lambda i: (0, i, 0))

    h_in = h
    h_fin = jnp.zeros((n, hid), _F32)
    for i in range(nl):
        xws_chunks = pl.pallas_call(
            _layer_mm_body,
            grid=(gn,),
            in_specs=[_rows(bn, hid), _full((hid, hid)), _deg_spec(bn)],
            out_specs=[_rows(bn, 16)] * 6,
            out_shape=[jax.ShapeDtypeStruct((n, 16), _F32)] * 6,
        )(h_in, params['conv_w'][i], deg_parts)

        agg_parts = _sc_agg(xws_chunks, src2d, dst2d, zer)

        h_in, h_fin = pl.pallas_call(
            _layer_post_body,
            grid=(gn,),
            in_specs=[_part_spec()] * 6 + [_rows(bn, 16)] * 6 +
                     [_deg_spec(bn), _rows(bn, hid), _rows(bn, hid),
                      _full((1, hid)), _full((1, hid)), _full((1, hid)),
                      _full((1, 1))],
            out_specs=[_rows(bn, hid), _rows(bn, hid)],
            out_shape=[jax.ShapeDtypeStruct((n, hid), _F32),
                       jax.ShapeDtypeStruct((n, hid), _F32)],
        )(*agg_parts, *xws_chunks, deg_parts, h_in, h_fin,
          _row2(params['conv_b'][i]), _row2(params['bn_g'][i]),
          _row2(params['bn_b'][i]), alpha[i].reshape(1, 1))

    reg, p_s, p_d = pl.pallas_call(
        _heads_body,
        grid=(gn,),
        in_specs=[_rows(bn, hid), _full((hid, hid)), _full((1, hid)),
                  _full((hid, 8)), _full((1, 8)), _full((hid, hid)),
                  _full((hid, hid))],
        out_specs=[_rows(bn, 8), _rows(bn, hid), _rows(bn, hid)],
        out_shape=[jax.ShapeDtypeStruct((n, 8), _F32),
                   jax.ShapeDtypeStruct((n, hid), _F32),
                   jax.ShapeDtypeStruct((n, hid), _F32)],
    )(h_fin, params['or_w1'], _row2(params['or_b1']), orw2, orb2, w_s, w_d)

    g1 = p_s[src]
    g2 = p_d[dst]

    scores8 = pl.pallas_call(
        _edge_score_body,
        grid=(ge,),
        in_specs=[_rows(be, hid), _rows(be, hid), _rows(be, hid),
                  _full((hid, 8)), _full((1, 8))],
        out_specs=_rows(be, 8),
        out_shape=jax.ShapeDtypeStruct((e, 8), _F32),
    )(g1, g2, e_contrib, ecw2, ecb2)

    edge_scores = scores8[:, 0]
    pred_offsets = reg[:, 0:2]
    pred_uncertainty = reg[:, 2:4]
    return (edge_scores, pred_offsets, pred_uncertainty, h_fin)


# speedup vs baseline: 4.5377x; 1.2539x over previous
import functools
import math

import jax
import jax.numpy as jnp
import numpy as np
from jax.experimental import pallas as pl
from jax.experimental.pallas import tpu as pltpu
from jax.experimental.pallas import tpu_sc as plsc

_F32 = jnp.float32


def _gelu(x):
    return x * 0.5 * (1.0 + jax.lax.erf(x * (2.0 ** -0.5)))


def _ln(x, g, b, eps=1e-5):
    m = jnp.mean(x, axis=-1, keepdims=True)
    v = jnp.mean((x - m) ** 2, axis=-1, keepdims=True)
    return (x - m) * jax.lax.rsqrt(v + eps) * g + b


def _row2(v):
    return v.reshape(1, -1).astype(_F32)


def _node_pre_body(x_ref, bmat_ref, w1x_ref, w1f_ref, b1_ref, lng_ref,
                   lnb_ref, w2_ref, b2_ref, h_ref):
    x = x_ref[...]
    bm = bmat_ref[...]
    xb = x.astype(jnp.bfloat16).astype(_F32)
    bmb = bm.astype(jnp.bfloat16).astype(_F32)
    proj = (2.0 * math.pi) * (xb[:, 0:1] * bmb[0:1, :] + xb[:, 1:2] * bmb[1:2, :])
    ff = jnp.concatenate([jnp.sin(proj), jnp.cos(proj)], axis=1)
    pre = (jnp.dot(x, w1x_ref[...], preferred_element_type=_F32)
           + jnp.dot(ff, w1f_ref[...], preferred_element_type=_F32)
           + b1_ref[...])
    t = _gelu(_ln(pre, lng_ref[...], lnb_ref[...]))
    h_ref[...] = jnp.dot(t, w2_ref[...], preferred_element_type=_F32) + b2_ref[...]


def _dinv_from_parts(deg_ref):
    deg = deg_ref[0, :, 0:1] + deg_ref[1, :, 0:1] + 1.0
    return jax.lax.rsqrt(jnp.maximum(deg, 1e-12))


def _layer_mm_body(h_ref, w_ref, deg_ref, *x_refs):
    xw = jnp.dot(h_ref[...], w_ref[...], preferred_element_type=_F32)
    dinv = _dinv_from_parts(deg_ref)
    xws = xw * dinv
    for k, xr in enumerate(x_refs):
        xr[...] = xws[:, 16 * k:16 * k + 16]


def _layer_post_body(*refs):
    a_refs = refs[0:6]
    x_refs = refs[6:12]
    (deg_ref, hin_ref, hfin_ref, b_ref, g_ref, bb_ref, alpha_ref,
     hout_ref, hfinout_ref) = refs[12:]
    dinv = _dinv_from_parts(deg_ref)
    agg = jnp.concatenate([a[0] + a[1] for a in a_refs], axis=-1)
    xws = jnp.concatenate([x[...] for x in x_refs], axis=-1)
    pre = (agg + xws) * dinv + b_ref[...]
    h_out = _gelu(_ln(pre, g_ref[...], bb_ref[...])) + hin_ref[...]
    hout_ref[...] = h_out
    hfinout_ref[...] = hfin_ref[...] + alpha_ref[0, 0] * h_out


_SC_NC = 2
_SC_NS = 16
_SC_RPS = 200
_SC_GRP = 5
_SC_EPAD = _SC_NC * _SC_NS * _SC_RPS * 128
_SC_NACC = 50176


def _sc_agg_body(t0, t1, t2, t3, t4, t5, src2d, dst2d, zer,
                 o0, o1, o2, o3, o4, o5,
                 srcidx, dstidx, buf0, buf1, accum, tloc,
                 semg0, semg1, sems0, sems1):
    cid = jax.lax.axis_index("c")
    sid = jax.lax.axis_index("s")
    rowbase = (cid * _SC_NS + sid) * _SC_RPS
    accrows = _SC_NACC // _SC_NS
    accbase = sid * accrows
    n_nodes = t0.shape[0]
    trows = n_nodes // _SC_NS
    tbase = sid * trows

    for table, out in ((t0, o0), (t1, o1), (t2, o2), (t3, o3), (t4, o4), (t5, o5)):
        pltpu.sync_copy(zer, accum.at[pl.ds(accbase, accrows)])
        pltpu.sync_copy(table.at[pl.ds(tbase, trows)],
                        tloc.at[pl.ds(tbase, trows)])
        plsc.subcore_barrier()

        def window(w, carry):
            pltpu.sync_copy(src2d.at[pl.ds(rowbase + w * (2 * _SC_GRP),
                                           2 * _SC_GRP)], srcidx)
            pltpu.sync_copy(dst2d.at[pl.ds(rowbase + w * (2 * _SC_GRP),
                                           2 * _SC_GRP)], dstidx)
            d0 = [pltpu.async_copy(tloc.at[srcidx.at[j]],
                                   buf0.at[pl.ds(j * 128, 128)], semg0)
                  for j in range(_SC_GRP)]
            for d in d0:
                d.wait()
            d1 = [pltpu.async_copy(tloc.at[srcidx.at[_SC_GRP + j]],
                                   buf1.at[pl.ds(j * 128, 128)], semg1)
                  for j in range(_SC_GRP)]
            s0 = [pltpu.async_copy(buf0.at[pl.ds(j * 128, 128)],
                                   accum.at[dstidx.at[j]], sems0,
                                   add=True)
                  for j in range(_SC_GRP)]
            for d in d1:
                d.wait()
            for d in s0:
                d.wait()
            s1 = [pltpu.async_copy(buf1.at[pl.ds(j * 128, 128)],
                                   accum.at[dstidx.at[_SC_GRP + j]],
                                   sems1, add=True)
                  for j in range(_SC_GRP)]
            for d in s1:
                d.wait()
            return carry

        jax.lax.fori_loop(0, _SC_RPS // (2 * _SC_GRP), window, 0)
        plsc.subcore_barrier()
        pltpu.sync_copy(accum.at[pl.ds(accbase, accrows)],
                        out.at[cid, pl.ds(accbase, accrows)])


def _sc_deg_body(dst2d, ones16, zer16, odeg, dstidx, buf, accum, semd):
    cid = jax.lax.axis_index("c")
    sid = jax.lax.axis_index("s")
    rowbase = (cid * _SC_NS + sid) * _SC_RPS
    accrows = _SC_NACC // _SC_NS
    accbase = sid * accrows

    pltpu.sync_copy(dst2d.at[pl.ds(rowbase, _SC_RPS)], dstidx)
    pltpu.sync_copy(ones16, buf)
    pltpu.sync_copy(zer16, accum.at[pl.ds(accbase, accrows)])
    plsc.subcore_barrier()

    def grp(i, carry):
        base0 = i * _SC_GRP
        ds = [pltpu.async_copy(buf.at[pl.ds(j * 128, 128)],
                               accum.at[dstidx.at[base0 + j]], semd,
                               add=True)
              for j in range(_SC_GRP)]
        for d in ds:
            d.wait()
        return carry

    jax.lax.fori_loop(0, _SC_RPS // _SC_GRP, grp, 0)
    plsc.subcore_barrier()
    pltpu.sync_copy(accum.at[pl.ds(accbase, accrows)],
                    odeg.at[cid, pl.ds(accbase, accrows)])


def _sc_deg(dst2d, ones16, zer16):
    mesh = plsc.VectorSubcoreMesh(core_axis_name="c", subcore_axis_name="s")
    f = pl.kernel(
        _sc_deg_body,
        out_type=jax.ShapeDtypeStruct((_SC_NC, _SC_NACC, 16), _F32),
        mesh=mesh,
        scratch_types=[
            pltpu.VMEM((_SC_RPS, 128), jnp.int32),
            pltpu.VMEM((_SC_GRP * 128, 16), _F32),
            pltpu.VMEM_SHARED((_SC_NACC, 16), _F32),
            pltpu.SemaphoreType.DMA,
        ],
        compiler_params=pltpu.CompilerParams(use_tc_tiling_on_sc=False),
    )
    return f(dst2d, ones16, zer16)


def _sc_agg(xws_chunks, src2d, dst2d, zer):
    n_acc = _SC_NACC
    n_nodes = xws_chunks[0].shape[0]
    mesh = plsc.VectorSubcoreMesh(core_axis_name="c", subcore_axis_name="s")
    f = pl.kernel(
        _sc_agg_body,
        out_type=[jax.ShapeDtypeStruct((_SC_NC, n_acc, 16), _F32)] * 6,
        mesh=mesh,
        scratch_types=[
            pltpu.VMEM((2 * _SC_GRP, 128), jnp.int32),
            pltpu.VMEM((2 * _SC_GRP, 128), jnp.int32),
            pltpu.VMEM((_SC_GRP * 128, 16), _F32),
            pltpu.VMEM((_SC_GRP * 128, 16), _F32),
            pltpu.VMEM_SHARED((n_acc, 16), _F32),
            pltpu.VMEM_SHARED((n_nodes, 16), _F32),
            pltpu.SemaphoreType.DMA,
            pltpu.SemaphoreType.DMA,
            pltpu.SemaphoreType.DMA,
            pltpu.SemaphoreType.DMA,
        ],
        compiler_params=pltpu.CompilerParams(use_tc_tiling_on_sc=False),
    )
    return f(*xws_chunks, src2d, dst2d, zer)


def _heads_body(hfin_ref, orw1_ref, orb1_ref, orw2_ref, orb2_ref, ws_ref,
                wd_ref, reg_ref, ps_ref, pd_ref):
    hf = hfin_ref[...]
    r = _gelu(jnp.dot(hf, orw1_ref[...], preferred_element_type=_F32) + orb1_ref[...])
    reg = jnp.dot(r, orw2_ref[...], preferred_element_type=_F32) + orb2_ref[...]
    off = jnp.clip(reg, -100.0, 100.0)
    unc = jnp.clip(jax.nn.softplus(reg), 0.01, 10.0)
    col = jax.lax.broadcasted_iota(jnp.int32, reg.shape, 1)
    reg_ref[...] = jnp.where(col < 2, off, unc)
    ps_ref[...] = jnp.dot(hf, ws_ref[...], preferred_element_type=_F32)
    pd_ref[...] = jnp.dot(hf, wd_ref[...], preferred_element_type=_F32)


def _edge_pre_body(ea_ref, w1_ref, b1_ref, lng_ref, lnb_ref, w2_ref, b2_ref,
                   we_ref, ce_ref, ec_ref):
    pre = jnp.dot(ea_ref[...], w1_ref[...], preferred_element_type=_F32) + b1_ref[...]
    t = _gelu(_ln(pre, lng_ref[...], lnb_ref[...]))
    ee = jnp.dot(t, w2_ref[...], preferred_element_type=_F32) + b2_ref[...]
    ec_ref[...] = jnp.dot(ee, we_ref[...], preferred_element_type=_F32) + ce_ref[...]


def _edge_score_body(g1_ref, g2_ref, ec_ref, w2_ref, b2_ref, out_ref):
    s = _gelu(g1_ref[...] + g2_ref[...] + ec_ref[...])
    logit = jnp.dot(s, w2_ref[...], preferred_element_type=_F32) + b2_ref[...]
    out_ref[...] = jax.nn.sigmoid(logit)


def _full(shape):
    return pl.BlockSpec(shape, lambda i: tuple(0 for _ in shape))


def _rows(bn, d):
    return pl.BlockSpec((bn, d), lambda i: (i, 0))


def kernel(x, edge_index, edge_attr, params):
    n, d_node = x.shape
    e = edge_attr.shape[0]
    hid = params['node_w2'].shape[0]
    nl = params['conv_w'].shape[0]
    src = edge_index[0]
    dst = edge_index[1]

    alpha = jax.nn.softmax(jnp.mean(params['attn_vector'], axis=1), axis=0)
    w_s = params['ec_w1'][0:hid]
    w_d = params['ec_w1'][hid:2 * hid]
    w_e = params['ec_w1'][2 * hid:3 * hid]
    c_e = params['ec_b1']
    bmat = jnp.zeros((8, 32), _F32).at[0:2].set(params['B'])
    w1x = params['node_w1'][:d_node]
    w1f = params['node_w1'][d_node:]
    orw2 = jnp.zeros((hid, 8), _F32).at[:, 0:4].set(params['or_w2'])
    orb2 = jnp.zeros((1, 8), _F32).at[0, 0:4].set(params['or_b2'])
    ecw2 = jnp.zeros((hid, 8), _F32).at[:, 0:1].set(params['ec_w2'])
    ecb2 = jnp.zeros((1, 8), _F32).at[0, 0:1].set(params['ec_b2'])

    bn = 1000
    be = 2000
    gn = n // bn
    ge = e // be

    npadrow = _SC_EPAD - e
    src2d = jnp.concatenate(
        [src, jnp.zeros((npadrow,), jnp.int32)]).reshape(-1, 128)
    dst2d = jnp.concatenate(
        [dst, jnp.full((npadrow,), n, jnp.int32)]).reshape(-1, 128)
    zer = jnp.zeros((_SC_NACC // _SC_NS, 16), _F32)
    zer16 = jnp.zeros((_SC_NACC // _SC_NS, 16), _F32)
    ones16 = jnp.zeros((_SC_GRP * 128, 16), _F32).at[:, 0].set(1.0)

    deg_parts = _sc_deg(dst2d, ones16, zer16)

    def _deg_spec(bn_):
        return pl.BlockSpec((_SC_NC, bn_, 16), lambda i: (0, i, 0))

    h = pl.pallas_call(
        _node_pre_body,
        grid=(gn,),
        in_specs=[_rows(bn, d_node), _full((8, 32)), _full((d_node, hid)),
                  _full((64, hid)), _full((1, hid)), _full((1, hid)),
                  _full((1, hid)), _full((hid, hid)), _full((1, hid))],
        out_specs=_rows(bn, hid),
        out_shape=jax.ShapeDtypeStruct((n, hid), _F32),
    )(x, bmat, w1x, w1f, _row2(params['node_b1']), _row2(params['node_ln_g']),
      _row2(params['node_ln_b']), params['node_w2'], _row2(params['node_b2']))

    e_contrib = pl.pallas_call(
        _edge_pre_body,
        grid=(ge,),
        in_specs=[_rows(be, 16), _full((16, hid)), _full((1, hid)),
                  _full((1, hid)), _full((1, hid)), _full((hid, hid)),
                  _full((1, hid)), _full((hid, hid)), _full((1, hid))],
        out_specs=_rows(be, hid),
        out_shape=jax.ShapeDtypeStruct((e, hid), _F32),
    )(edge_attr, params['edge_w1'], _row2(params['edge_b1']),
      _row2(params['edge_ln_g']), _row2(params['edge_ln_b']),
      params['edge_w2'], _row2(params['edge_b2']), w_e, _row2(c_e))

    def _part_spec():
        return pl.BlockSpec((_SC_NC, bn, 16), lambda i: (0, i, 0))

    h_in = h
    h_fin = jnp.zeros((n, hid), _F32)
    for i in range(nl):
        xws_chunks = pl.pallas_call(
            _layer_mm_body,
            grid=(gn,),
            in_specs=[_rows(bn, hid), _full((hid, hid)), _deg_spec(bn)],
            out_specs=[_rows(bn, 16)] * 6,
            out_shape=[jax.ShapeDtypeStruct((n, 16), _F32)] * 6,
        )(h_in, params['conv_w'][i], deg_parts)

        agg_parts = _sc_agg(xws_chunks, src2d, dst2d, zer)

        h_in, h_fin = pl.pallas_call(
            _layer_post_body,
            grid=(gn,),
            in_specs=[_part_spec()] * 6 + [_rows(bn, 16)] * 6 +
                     [_deg_spec(bn), _rows(bn, hid), _rows(bn, hid),
                      _full((1, hid)), _full((1, hid)), _full((1, hid)),
                      _full((1, 1))],
            out_specs=[_rows(bn, hid), _rows(bn, hid)],
            out_shape=[jax.ShapeDtypeStruct((n, hid), _F32),
                       jax.ShapeDtypeStruct((n, hid), _F32)],
        )(*agg_parts, *xws_chunks, deg_parts, h_in, h_fin,
          _row2(params['conv_b'][i]), _row2(params['bn_g'][i]),
          _row2(params['bn_b'][i]), alpha[i].reshape(1, 1))

    reg, p_s, p_d = pl.pallas_call(
        _heads_body,
        grid=(gn,),
        in_specs=[_rows(bn, hid), _full((hid, hid)), _full((1, hid)),
                  _full((hid, 8)), _full((1, 8)), _full((hid, hid)),
                  _full((hid, hid))],
        out_specs=[_rows(bn, 8), _rows(bn, hid), _rows(bn, hid)],
        out_shape=[jax.ShapeDtypeStruct((n, 8), _F32),
                   jax.ShapeDtypeStruct((n, hid), _F32),
                   jax.ShapeDtypeStruct((n, hid), _F32)],
    )(h_fin, params['or_w1'], _row2(params['or_b1']), orw2, orb2, w_s, w_d)

    g1 = p_s[src]
    g2 = p_d[dst]

    scores8 = pl.pallas_call(
        _edge_score_body,
        grid=(ge,),
        in_specs=[_rows(be, hid), _rows(be, hid), _rows(be, hid),
                  _full((hid, 8)), _full((1, 8))],
        out_specs=_rows(be, 8),
        out_shape=jax.ShapeDtypeStruct((e, 8), _F32),
    )(g1, g2, e_contrib, ecw2, ecb2)

    edge_scores = scores8[:, 0]
    pred_offsets = reg[:, 0:2]
    pred_uncertainty = reg[:, 2:4]
    return (edge_scores, pred_offsets, pred_uncertainty, h_fin)
